# baseline (speedup 1.0000x reference)
import functools

import jax
import jax.numpy as jnp
from jax.experimental import pallas as pl
from jax.experimental.pallas import tpu as pltpu
from jax.experimental.pallas import tpu_sc as plsc

TEMP = 0.01
COMMIT = 0.25


def _vq_body(nsteps, total_tokens, z_ref, ct_ref,
             idx_ref, ent_ref, emb_ref, com_ref,
             accp_ref, accpl_ref, accsq_ref):
    i = pl.program_id(0)
    z = z_ref[...]
    ct = ct_ref[...]
    n = ct.shape[1]
    zsq = jnp.sum(z * z, axis=1, keepdims=True)
    csq = jnp.sum(ct * ct, axis=0, keepdims=True)
    dots = jax.lax.dot_general(
        z, ct, (((1,), (0,)), ((), ())),
        preferred_element_type=jnp.float32,
        precision=jax.lax.Precision.DEFAULT)
    d = zsq - 2.0 * dots + csq

    dmin = jnp.min(d, axis=1, keepdims=True)
    lane = jax.lax.broadcasted_iota(jnp.int32, d.shape, 1)
    idx = jnp.min(jnp.where(d == dmin, lane, n), axis=1)
    idx_ref[...] = idx.reshape(idx_ref.shape)

    sq = jnp.sum(dmin, keepdims=True).reshape(1, 1)

    aff = (-d) / TEMP
    m = jnp.max(aff, axis=1, keepdims=True)
    sh = aff - m
    e = jnp.exp(sh)
    s = jnp.sum(e, axis=1, keepdims=True)
    probs = e / s
    logp = sh - jnp.log(s)
    plsum = jnp.sum(probs * logp, keepdims=True).reshape(1, 1)
    colsum = jnp.sum(probs, axis=0, keepdims=True)

    @pl.when(i == 0)
    def _():
        accp_ref[...] = colsum
        accpl_ref[...] = plsum
        accsq_ref[...] = sq

    @pl.when(i > 0)
    def _():
        accp_ref[...] += colsum
        accpl_ref[...] += plsum
        accsq_ref[...] += sq

    @pl.when(i == nsteps - 1)
    def _():
        tt = jnp.float32(total_tokens)
        avg_probs = accp_ref[...] / tt
        avg_entropy = -jnp.sum(avg_probs * jnp.log(avg_probs + 1e-5),
                               keepdims=True).reshape(1, 1)
        sample_entropy = -(accpl_ref[...] / tt)
        ent_ref[...] = 0.1 * (sample_entropy - avg_entropy)
        msq = accsq_ref[...] / (tt * z.shape[1])
        emb_ref[...] = msq
        com_ref[...] = COMMIT * msq


def _sc_gather(codebook, idx_flat):
    b = idx_flat.shape[0]
    n, dd = codebook.shape
    nc, ns = 2, 16
    nw = nc * ns
    bw = b // nw
    chunk = 128
    mesh = plsc.VectorSubcoreMesh(core_axis_name="c", subcore_axis_name="s")

    @functools.partial(
        pl.kernel, mesh=mesh,
        out_type=jax.ShapeDtypeStruct((b, dd), jnp.float32),
        scratch_types=[pltpu.VMEM((bw,), jnp.int32),
                       pltpu.VMEM((bw, dd), jnp.float32),
                       pltpu.SemaphoreType.DMA],
        compiler_params=pltpu.CompilerParams(use_tc_tiling_on_sc=False),
    )
    def k(table_hbm, idx_hbm, out_hbm, idx_v, rows_v, sem):
        wid = jax.lax.axis_index("s") * nc + jax.lax.axis_index("c")
        base = wid * bw
        pltpu.sync_copy(idx_hbm.at[pl.ds(base, bw)], idx_v)
        copies = []
        for j in range(bw // chunk):
            copies.append(pltpu.async_copy(
                table_hbm.at[idx_v.at[pl.ds(j * chunk, chunk)]],
                rows_v.at[pl.ds(j * chunk, chunk)], sem))
        for c in copies:
            c.wait()
        pltpu.sync_copy(rows_v, out_hbm.at[pl.ds(base, bw)])

    return k(codebook, idx_flat)


def kernel(z_e, codebook):
    codebook = jnp.asarray(codebook, dtype=jnp.float32)
    n, d = codebook.shape
    z_flat = jnp.reshape(z_e, (-1, d)).astype(jnp.float32)
    t = z_flat.shape[0]
    bt = 1024
    nsteps = t // bt
    ct = codebook.T

    out_shapes = (
        jax.ShapeDtypeStruct((nsteps, 1, bt), jnp.int32),
        jax.ShapeDtypeStruct((1, 1), jnp.float32),
        jax.ShapeDtypeStruct((1, 1), jnp.float32),
        jax.ShapeDtypeStruct((1, 1), jnp.float32),
    )
    idx3, ent, emb, com = pl.pallas_call(
        functools.partial(_vq_body, nsteps, t),
        grid=(nsteps,),
        in_specs=[
            pl.BlockSpec((bt, d), lambda i: (i, 0)),
            pl.BlockSpec((d, n), lambda i: (0, 0)),
        ],
        out_specs=[
            pl.BlockSpec((1, 1, bt), lambda i: (i, 0, 0)),
            pl.BlockSpec((1, 1), lambda i: (0, 0)),
            pl.BlockSpec((1, 1), lambda i: (0, 0)),
            pl.BlockSpec((1, 1), lambda i: (0, 0)),
        ],
        out_shape=out_shapes,
        scratch_shapes=[
            pltpu.VMEM((1, n), jnp.float32),
            pltpu.VMEM((1, 1), jnp.float32),
            pltpu.VMEM((1, 1), jnp.float32),
        ],
    )(z_flat, ct)

    encoding_indices = idx3.reshape(t)
    quantized = _sc_gather(codebook, encoding_indices).reshape(z_e.shape)
    return (quantized, com.reshape(()), emb.reshape(()),
            ent.reshape(()), encoding_indices)


# speedup vs baseline: 1.2757x; 1.2757x over previous
import functools

import jax
import jax.numpy as jnp
from jax.experimental import pallas as pl
from jax.experimental.pallas import tpu as pltpu
from jax.experimental.pallas import tpu_sc as plsc

TEMP = 0.01
COMMIT = 0.25


def _vq_body(nsteps, total_tokens, z_ref, ct_ref,
             idx_ref, ent_ref, emb_ref, com_ref,
             accp_ref, accpl_ref, accsq_ref):
    i = pl.program_id(0)
    z = z_ref[...]
    ct = ct_ref[...]
    n = ct.shape[1]
    zsq = jnp.sum(z * z, axis=1, keepdims=True)
    csq = jnp.sum(ct * ct, axis=0, keepdims=True)
    dots = jax.lax.dot_general(
        z, ct, (((1,), (0,)), ((), ())),
        preferred_element_type=jnp.float32,
        precision=jax.lax.Precision.DEFAULT)
    d = zsq - 2.0 * dots + csq

    dmin = jnp.min(d, axis=1, keepdims=True)
    lane = jax.lax.broadcasted_iota(jnp.int32, d.shape, 1)
    idx = jnp.min(jnp.where(d == dmin, lane, n), axis=1)
    idx_ref[...] = idx.reshape(idx_ref.shape)

    sq = jnp.sum(dmin, keepdims=True).reshape(1, 1)

    sh = (dmin - d) * (1.0 / TEMP)
    e = jnp.exp(sh)
    s = jnp.sum(e, axis=1, keepdims=True)
    es_sum = jnp.sum(e * sh, axis=1, keepdims=True)
    rcp_s = 1.0 / s
    probs = e * rcp_s
    colsum = jnp.sum(probs, axis=0, keepdims=True)
    plrow = es_sum * rcp_s - jnp.log(s) * (s * rcp_s)
    plsum = jnp.sum(plrow, keepdims=True).reshape(1, 1)

    @pl.when(i == 0)
    def _():
        accp_ref[...] = colsum
        accpl_ref[...] = plsum
        accsq_ref[...] = sq

    @pl.when(i > 0)
    def _():
        accp_ref[...] += colsum
        accpl_ref[...] += plsum
        accsq_ref[...] += sq

    @pl.when(i == nsteps - 1)
    def _():
        tt = jnp.float32(total_tokens)
        avg_probs = accp_ref[...] / tt
        avg_entropy = -jnp.sum(avg_probs * jnp.log(avg_probs + 1e-5),
                               keepdims=True).reshape(1, 1)
        sample_entropy = -(accpl_ref[...] / tt)
        ent_ref[...] = 0.1 * (sample_entropy - avg_entropy)
        msq = accsq_ref[...] / (tt * z.shape[1])
        emb_ref[...] = msq
        com_ref[...] = COMMIT * msq


def _sc_gather(codebook, idx_flat):
    b = idx_flat.shape[0]
    n, dd = codebook.shape
    nc, ns = 2, 16
    nw = nc * ns
    bw = b // nw
    chunk = 128
    mesh = plsc.VectorSubcoreMesh(core_axis_name="c", subcore_axis_name="s")

    @functools.partial(
        pl.kernel, mesh=mesh,
        out_type=jax.ShapeDtypeStruct((b, dd), jnp.float32),
        scratch_types=[pltpu.VMEM((bw,), jnp.int32),
                       pltpu.VMEM((bw, dd), jnp.float32),
                       pltpu.SemaphoreType.DMA],
        compiler_params=pltpu.CompilerParams(use_tc_tiling_on_sc=False),
    )
    def k(table_hbm, idx_hbm, out_hbm, idx_v, rows_v, sem):
        wid = jax.lax.axis_index("s") * nc + jax.lax.axis_index("c")
        base = wid * bw
        pltpu.sync_copy(idx_hbm.at[pl.ds(base, bw)], idx_v)
        copies = []
        for j in range(bw // chunk):
            copies.append(pltpu.async_copy(
                table_hbm.at[idx_v.at[pl.ds(j * chunk, chunk)]],
                rows_v.at[pl.ds(j * chunk, chunk)], sem))
        for c in copies:
            c.wait()
        pltpu.sync_copy(rows_v, out_hbm.at[pl.ds(base, bw)])

    return k(codebook, idx_flat)


def kernel(z_e, codebook):
    codebook = jnp.asarray(codebook, dtype=jnp.float32)
    n, d = codebook.shape
    z_flat = jnp.reshape(z_e, (-1, d)).astype(jnp.float32)
    t = z_flat.shape[0]
    bt = 2048
    nsteps = t // bt
    ct = codebook.T

    out_shapes = (
        jax.ShapeDtypeStruct((nsteps, 1, bt), jnp.int32),
        jax.ShapeDtypeStruct((1, 1), jnp.float32),
        jax.ShapeDtypeStruct((1, 1), jnp.float32),
        jax.ShapeDtypeStruct((1, 1), jnp.float32),
    )
    idx3, ent, emb, com = pl.pallas_call(
        functools.partial(_vq_body, nsteps, t),
        grid=(nsteps,),
        in_specs=[
            pl.BlockSpec((bt, d), lambda i: (i, 0)),
            pl.BlockSpec((d, n), lambda i: (0, 0)),
        ],
        out_specs=[
            pl.BlockSpec((1, 1, bt), lambda i: (i, 0, 0)),
            pl.BlockSpec((1, 1), lambda i: (0, 0)),
            pl.BlockSpec((1, 1), lambda i: (0, 0)),
            pl.BlockSpec((1, 1), lambda i: (0, 0)),
        ],
        out_shape=out_shapes,
        scratch_shapes=[
            pltpu.VMEM((1, n), jnp.float32),
            pltpu.VMEM((1, 1), jnp.float32),
            pltpu.VMEM((1, 1), jnp.float32),
        ],
    )(z_flat, ct)

    encoding_indices = idx3.reshape(t)
    quantized = _sc_gather(codebook, encoding_indices).reshape(z_e.shape)
    return (quantized, com.reshape(()), emb.reshape(()),
            ent.reshape(()), encoding_indices)


# speedup vs baseline: 1.2883x; 1.0099x over previous
import functools

import jax
import jax.numpy as jnp
from jax.experimental import pallas as pl
from jax.experimental.pallas import tpu as pltpu
from jax.experimental.pallas import tpu_sc as plsc

TEMP = 0.01
COMMIT = 0.25


def _vq_body(nsteps, total_tokens, z_ref, ct_ref,
             idx_ref, ent_ref, emb_ref, com_ref,
             accp_ref, accpl_ref, accsq_ref):
    i = pl.program_id(0)
    z = z_ref[...]
    ct = ct_ref[...]
    n = ct.shape[1]
    zsq = jnp.sum(z * z, axis=1, keepdims=True)
    csq = jnp.sum(ct * ct, axis=0, keepdims=True)
    dots = jax.lax.dot_general(
        z, ct, (((1,), (0,)), ((), ())),
        preferred_element_type=jnp.float32,
        precision=jax.lax.Precision.DEFAULT)
    d = zsq - 2.0 * dots + csq

    dmin = jnp.min(d, axis=1, keepdims=True)
    lane = jax.lax.broadcasted_iota(jnp.int32, d.shape, 1)
    idx = jnp.min(jnp.where(d == dmin, lane, n), axis=1)
    idx_ref[...] = idx.reshape(idx_ref.shape)

    sq = jnp.sum(dmin, keepdims=True).reshape(1, 1)

    sh = (dmin - d) * (1.0 / TEMP)
    e = jnp.exp(sh)
    s = jnp.sum(e, axis=1, keepdims=True)
    es_sum = jnp.sum(e * sh, axis=1, keepdims=True)
    rcp_s = 1.0 / s
    probs = e * rcp_s
    colsum = jnp.sum(probs, axis=0, keepdims=True)
    plrow = es_sum * rcp_s - jnp.log(s) * (s * rcp_s)
    plsum = jnp.sum(plrow, keepdims=True).reshape(1, 1)

    @pl.when(i == 0)
    def _():
        accp_ref[...] = colsum
        accpl_ref[...] = plsum
        accsq_ref[...] = sq

    @pl.when(i > 0)
    def _():
        accp_ref[...] += colsum
        accpl_ref[...] += plsum
        accsq_ref[...] += sq

    @pl.when(i == nsteps - 1)
    def _():
        tt = jnp.float32(total_tokens)
        avg_probs = accp_ref[...] / tt
        avg_entropy = -jnp.sum(avg_probs * jnp.log(avg_probs + 1e-5),
                               keepdims=True).reshape(1, 1)
        sample_entropy = -(accpl_ref[...] / tt)
        ent_ref[...] = 0.1 * (sample_entropy - avg_entropy)
        msq = accsq_ref[...] / (tt * z.shape[1])
        emb_ref[...] = msq
        com_ref[...] = COMMIT * msq


def _sc_gather(codebook, idx_flat):
    b = idx_flat.shape[0]
    n, dd = codebook.shape
    nc, ns = 2, 16
    nw = nc * ns
    bw = b // nw
    chunk = 128
    mesh = plsc.VectorSubcoreMesh(core_axis_name="c", subcore_axis_name="s")

    @functools.partial(
        pl.kernel, mesh=mesh,
        out_type=jax.ShapeDtypeStruct((b, dd), jnp.float32),
        scratch_types=[pltpu.VMEM((bw,), jnp.int32),
                       pltpu.VMEM((bw, dd), jnp.float32),
                       pltpu.SemaphoreType.DMA],
        compiler_params=pltpu.CompilerParams(use_tc_tiling_on_sc=False),
    )
    def k(table_hbm, idx_hbm, out_hbm, idx_v, rows_v, sem):
        wid = jax.lax.axis_index("s") * nc + jax.lax.axis_index("c")
        base = wid * bw
        pltpu.sync_copy(idx_hbm.at[pl.ds(base, bw)], idx_v)
        copies = []
        for j in range(bw // chunk):
            copies.append(pltpu.async_copy(
                table_hbm.at[idx_v.at[pl.ds(j * chunk, chunk)]],
                rows_v.at[pl.ds(j * chunk, chunk)], sem))
        for c in copies:
            c.wait()
        pltpu.sync_copy(rows_v, out_hbm.at[pl.ds(base, bw)])

    return k(codebook, idx_flat)


def kernel(z_e, codebook):
    codebook = jnp.asarray(codebook, dtype=jnp.float32)
    n, d = codebook.shape
    z_flat = jnp.reshape(z_e, (-1, d)).astype(jnp.float32)
    t = z_flat.shape[0]
    bt = 4096
    nsteps = t // bt
    ct = codebook.T

    out_shapes = (
        jax.ShapeDtypeStruct((nsteps, 1, bt), jnp.int32),
        jax.ShapeDtypeStruct((1, 1), jnp.float32),
        jax.ShapeDtypeStruct((1, 1), jnp.float32),
        jax.ShapeDtypeStruct((1, 1), jnp.float32),
    )
    idx3, ent, emb, com = pl.pallas_call(
        functools.partial(_vq_body, nsteps, t),
        grid=(nsteps,),
        in_specs=[
            pl.BlockSpec((bt, d), lambda i: (i, 0)),
            pl.BlockSpec((d, n), lambda i: (0, 0)),
        ],
        out_specs=[
            pl.BlockSpec((1, 1, bt), lambda i: (i, 0, 0)),
            pl.BlockSpec((1, 1), lambda i: (0, 0)),
            pl.BlockSpec((1, 1), lambda i: (0, 0)),
            pl.BlockSpec((1, 1), lambda i: (0, 0)),
        ],
        out_shape=out_shapes,
        scratch_shapes=[
            pltpu.VMEM((1, n), jnp.float32),
            pltpu.VMEM((1, 1), jnp.float32),
            pltpu.VMEM((1, 1), jnp.float32),
        ],
    )(z_flat, ct)

    encoding_indices = idx3.reshape(t)
    quantized = _sc_gather(codebook, encoding_indices).reshape(z_e.shape)
    return (quantized, com.reshape(()), emb.reshape(()),
            ent.reshape(()), encoding_indices)
